# baseline (speedup 1.0000x reference)
import jax
import jax.numpy as jnp
from jax import lax
from jax.experimental import pallas as pl
from jax.experimental.pallas import tpu as pltpu
from jax.experimental.pallas import tpu_sc as plsc

_VOCAB = 50257
_BATCH = 4
_SEQ = 4096
_K = 2
_NC = 2
_NS = 16
_NW = _NC * _NS
_CHUNK = 128
_NSB = _SEQ // _CHUNK


def _router_body(ids_hbm, t0_hbm, t1_hbm, out_hbm, ids_v, g0_v, g1_v, sem):
    wid = lax.axis_index("s") * _NC + lax.axis_index("c")
    pltpu.sync_copy(ids_hbm.at[wid], ids_v)
    copies = []
    for j in range(_BATCH):
        copies.append(pltpu.async_copy(t0_hbm.at[ids_v.at[j]], g0_v.at[j], sem))
        copies.append(pltpu.async_copy(t1_hbm.at[ids_v.at[j]], g1_v.at[j], sem))
    for c in copies:
        c.wait()
    pltpu.sync_copy(g0_v, out_hbm.at[:, wid, 0])
    pltpu.sync_copy(g1_v, out_hbm.at[:, wid, 1])


@jax.jit
def _route(ids3, t0, t1):
    mesh = plsc.VectorSubcoreMesh(
        core_axis_name="c", subcore_axis_name="s", num_cores=_NC,
        num_subcores=_NS,
    )
    call = pl.kernel(
        _router_body,
        out_type=jax.ShapeDtypeStruct((_BATCH, _NSB, _K, _CHUNK), jnp.int32),
        mesh=mesh,
        scratch_types=[
            pltpu.VMEM((_BATCH, _CHUNK), jnp.int32),
            pltpu.VMEM((_BATCH, _CHUNK), jnp.int32),
            pltpu.VMEM((_BATCH, _CHUNK), jnp.int32),
            pltpu.SemaphoreType.DMA,
        ],
        compiler_params=pltpu.CompilerParams(
            use_tc_tiling_on_sc=False, needs_layout_passes=False,
        ),
    )
    return call(ids3, t0, t1)


def kernel(input, hash_table):
    ids3 = input.astype(jnp.int32).reshape(_BATCH, _NSB, _CHUNK).transpose(1, 0, 2)
    t0 = hash_table[:, 0]
    t1 = hash_table[:, 1]
    out = _route(ids3, t0, t1)
    return out.transpose(0, 1, 3, 2).reshape(_BATCH, _SEQ, _K)
